# baseline (speedup 1.0000x reference)
import functools

import jax
import jax.numpy as jnp
from jax.experimental import pallas as pl
from jax.experimental.pallas import tpu as pltpu

_BC = 128
_NBUF = 32
_NCORES = 2


def _compute(lam_ref, xv, g):
    lx = jnp.log(xv)
    lam = jnp.full(xv.shape, lam_ref[0], dtype=jnp.float32)
    inv = jnp.full(xv.shape, 1.0 / lam_ref[0], dtype=jnp.float32)
    for k in range(1, 8):
        m = g == (k + 1)
        lam = jnp.where(m, lam_ref[k], lam)
        inv = jnp.where(m, 1.0 / lam_ref[k], inv)
    pow_branch = (jnp.exp(lam * lx) - 1.0) * inv
    return jnp.where(jnp.abs(lam) < 0.01, lx, pow_branch)


def kernel(x, group, lmbdas):
    R, C = x.shape
    xt = x.T
    gt = group.T

    cols_per_core = R // _NCORES
    nchunks = cols_per_core // _BC
    ngroups = nchunks // _NBUF
    mesh = pltpu.create_tensorcore_mesh("core", num_cores=_NCORES)

    @functools.partial(
        pl.kernel,
        out_type=jax.ShapeDtypeStruct((C, R), jnp.float32),
        mesh=mesh,
        scratch_types=[
            pltpu.SMEM((8,), jnp.float32),
            pltpu.VMEM((_NBUF, C, _BC), jnp.float32),
            pltpu.VMEM((_NBUF, C, _BC), jnp.int32),
            pltpu.VMEM((_NBUF, C, _BC), jnp.float32),
            pltpu.SemaphoreType.DMA((_NBUF,)),
            pltpu.SemaphoreType.DMA((_NBUF,)),
            pltpu.SemaphoreType.DMA((_NBUF,)),
        ],
    )
    def run(lam_hbm, x_hbm, g_hbm, o_hbm, lam_s, xb, gb, ob, xs, gs, osem):
        core = jax.lax.axis_index("core")
        col0 = core * cols_per_core

        pltpu.sync_copy(lam_hbm, lam_s)

        def start_in(b, c):
            s = col0 + c * _BC
            pltpu.make_async_copy(x_hbm.at[:, pl.ds(s, _BC)], xb.at[b], xs.at[b]).start()
            pltpu.make_async_copy(g_hbm.at[:, pl.ds(s, _BC)], gb.at[b], gs.at[b]).start()

        def wait_in(b, c):
            s = col0 + c * _BC
            pltpu.make_async_copy(x_hbm.at[:, pl.ds(s, _BC)], xb.at[b], xs.at[b]).wait()
            pltpu.make_async_copy(g_hbm.at[:, pl.ds(s, _BC)], gb.at[b], gs.at[b]).wait()

        def start_out(b, c):
            s = col0 + c * _BC
            pltpu.make_async_copy(ob.at[b], o_hbm.at[:, pl.ds(s, _BC)], osem.at[b]).start()

        def wait_out(b, c):
            s = col0 + c * _BC
            pltpu.make_async_copy(ob.at[b], o_hbm.at[:, pl.ds(s, _BC)], osem.at[b]).wait()

        for b in range(_NBUF):
            start_in(b, b)

        def gbody(gi, carry):
            for b in range(_NBUF):
                c = gi * _NBUF + b
                wait_in(b, c)

                @pl.when(gi > 0)
                def _():
                    wait_out(b, c - _NBUF)

                ob[b] = _compute(lam_s, xb[b], gb[b])
                start_out(b, c)

                @pl.when(gi + 1 < ngroups)
                def _():
                    start_in(b, c + _NBUF)

            return carry

        jax.lax.fori_loop(0, ngroups, gbody, 0)

        for b in range(_NBUF):
            wait_out(b, (ngroups - 1) * _NBUF + b)

    return run(lmbdas, xt, gt).T


# speedup vs baseline: 1.0188x; 1.0188x over previous
import functools

import jax
import jax.numpy as jnp
from jax.experimental import pallas as pl
from jax.experimental.pallas import tpu as pltpu

_BC = 128
_NBUF = 16
_NCORES = 2


def _compute(lam_ref, xv, g):
    lx = jnp.log(xv)
    lam = jnp.full(xv.shape, lam_ref[0], dtype=jnp.float32)
    inv = jnp.full(xv.shape, 1.0 / lam_ref[0], dtype=jnp.float32)
    for k in range(1, 8):
        m = g == (k + 1)
        lam = jnp.where(m, lam_ref[k], lam)
        inv = jnp.where(m, 1.0 / lam_ref[k], inv)
    pow_branch = (jnp.exp(lam * lx) - 1.0) * inv
    return jnp.where(jnp.abs(lam) < 0.01, lx, pow_branch)


def kernel(x, group, lmbdas):
    R, C = x.shape
    xt = x.T
    gt = group.T

    cols_per_core = R // _NCORES
    nchunks = cols_per_core // _BC
    ngroups = nchunks // _NBUF
    mesh = pltpu.create_tensorcore_mesh("core", num_cores=_NCORES)

    @functools.partial(
        pl.kernel,
        out_type=jax.ShapeDtypeStruct((C, R), jnp.float32),
        mesh=mesh,
        scratch_types=[
            pltpu.SMEM((8,), jnp.float32),
            pltpu.VMEM((_NBUF, C, _BC), jnp.float32),
            pltpu.VMEM((_NBUF, C, _BC), jnp.int32),
            pltpu.VMEM((_NBUF, C, _BC), jnp.float32),
            pltpu.SemaphoreType.DMA((_NBUF,)),
            pltpu.SemaphoreType.DMA((_NBUF,)),
            pltpu.SemaphoreType.DMA((_NBUF,)),
        ],
    )
    def run(lam_hbm, x_hbm, g_hbm, o_hbm, lam_s, xb, gb, ob, xs, gs, osem):
        core = jax.lax.axis_index("core")
        col0 = core * cols_per_core

        pltpu.sync_copy(lam_hbm, lam_s)

        def start_in(b, c):
            s = col0 + c * _BC
            pltpu.make_async_copy(x_hbm.at[:, pl.ds(s, _BC)], xb.at[b], xs.at[b]).start()
            pltpu.make_async_copy(g_hbm.at[:, pl.ds(s, _BC)], gb.at[b], gs.at[b]).start()

        def wait_in(b, c):
            s = col0 + c * _BC
            pltpu.make_async_copy(x_hbm.at[:, pl.ds(s, _BC)], xb.at[b], xs.at[b]).wait()
            pltpu.make_async_copy(g_hbm.at[:, pl.ds(s, _BC)], gb.at[b], gs.at[b]).wait()

        def start_out(b, c):
            s = col0 + c * _BC
            pltpu.make_async_copy(ob.at[b], o_hbm.at[:, pl.ds(s, _BC)], osem.at[b]).start()

        def wait_out(b, c):
            s = col0 + c * _BC
            pltpu.make_async_copy(ob.at[b], o_hbm.at[:, pl.ds(s, _BC)], osem.at[b]).wait()

        for b in range(_NBUF):
            start_in(b, b)

        def gbody(gi, carry):
            for b in range(_NBUF):
                c = gi * _NBUF + b
                wait_in(b, c)

                @pl.when(gi > 0)
                def _():
                    wait_out(b, c - _NBUF)

                ob[b] = _compute(lam_s, xb[b], gb[b])
                start_out(b, c)

                @pl.when(gi + 1 < ngroups)
                def _():
                    start_in(b, c + _NBUF)

            return carry

        jax.lax.fori_loop(0, ngroups, gbody, 0)

        for b in range(_NBUF):
            wait_out(b, (ngroups - 1) * _NBUF + b)

    return run(lmbdas, xt, gt).T


# speedup vs baseline: 1.1084x; 1.0879x over previous
import jax
import jax.numpy as jnp
from jax.experimental import pallas as pl
from jax.experimental.pallas import tpu as pltpu

_BC = 128
_NBUF = 32


def _compute(lam_ref, xv, g):
    lx = jnp.log(xv)
    lam = jnp.full(xv.shape, lam_ref[0], dtype=jnp.float32)
    inv = jnp.full(xv.shape, 1.0 / lam_ref[0], dtype=jnp.float32)
    for k in range(1, 8):
        m = g == (k + 1)
        lam = jnp.where(m, lam_ref[k], lam)
        inv = jnp.where(m, 1.0 / lam_ref[k], inv)
    pow_branch = (jnp.exp(lam * lx) - 1.0) * inv
    return jnp.where(jnp.abs(lam) < 0.01, lx, pow_branch)


def _make_body(R, C):
    nchunks = C // _BC
    ngroups = nchunks // _NBUF

    def body(lam_ref, x_hbm, g_hbm, o_hbm, xb, gb, ob, xs, gs, osem):
        def start_in(b, c):
            pltpu.make_async_copy(x_hbm.at[:, pl.ds(c * _BC, _BC)], xb.at[b], xs.at[b]).start()
            pltpu.make_async_copy(g_hbm.at[:, pl.ds(c * _BC, _BC)], gb.at[b], gs.at[b]).start()

        def wait_in(b, c):
            pltpu.make_async_copy(x_hbm.at[:, pl.ds(c * _BC, _BC)], xb.at[b], xs.at[b]).wait()
            pltpu.make_async_copy(g_hbm.at[:, pl.ds(c * _BC, _BC)], gb.at[b], gs.at[b]).wait()

        def start_out(b, c):
            pltpu.make_async_copy(ob.at[b], o_hbm.at[:, pl.ds(c * _BC, _BC)], osem.at[b]).start()

        def wait_out(b, c):
            pltpu.make_async_copy(ob.at[b], o_hbm.at[:, pl.ds(c * _BC, _BC)], osem.at[b]).wait()

        for b in range(_NBUF):
            start_in(b, b)

        def gbody(gi, carry):
            for b in range(_NBUF):
                c = gi * _NBUF + b
                wait_in(b, c)

                @pl.when(gi > 0)
                def _():
                    wait_out(b, c - _NBUF)

                ob[b] = _compute(lam_ref, xb[b], gb[b])
                start_out(b, c)

                @pl.when(gi + 1 < ngroups)
                def _():
                    start_in(b, c + _NBUF)

            return carry

        jax.lax.fori_loop(0, ngroups, gbody, 0)

        for b in range(_NBUF):
            wait_out(b, (ngroups - 1) * _NBUF + b)

    return body


def kernel(x, group, lmbdas):
    R, C = x.shape
    xt = x.T
    gt = group.T
    out_t = pl.pallas_call(
        _make_body(C, R),
        in_specs=[
            pl.BlockSpec(memory_space=pltpu.SMEM),
            pl.BlockSpec(memory_space=pltpu.HBM),
            pl.BlockSpec(memory_space=pltpu.HBM),
        ],
        out_specs=pl.BlockSpec(memory_space=pltpu.HBM),
        out_shape=jax.ShapeDtypeStruct((C, R), jnp.float32),
        scratch_shapes=[
            pltpu.VMEM((_NBUF, C, _BC), jnp.float32),
            pltpu.VMEM((_NBUF, C, _BC), jnp.int32),
            pltpu.VMEM((_NBUF, C, _BC), jnp.float32),
            pltpu.SemaphoreType.DMA((_NBUF,)),
            pltpu.SemaphoreType.DMA((_NBUF,)),
            pltpu.SemaphoreType.DMA((_NBUF,)),
        ],
    )(lmbdas, xt, gt)
    return out_t.T
